# baseline (speedup 1.0000x reference)
import functools

import jax
import jax.numpy as jnp
from jax import lax
from jax.experimental import pallas as pl
from jax.experimental.pallas import tpu as pltpu
from jax.experimental.pallas import tpu_sc as plsc

_NC = 2
_NS = 16
_NW = _NC * _NS
_CH = 128


def _sc_segment_sum(src2d, dst2d, feat, zeros, n_pad, f, k, ch,
                    interpret=False):
  mesh = plsc.VectorSubcoreMesh(core_axis_name="c", subcore_axis_name="s",
                                num_cores=_NC, num_subcores=_NS)
  rpt = n_pad // _NS

  def body(src_hbm, dst_hbm, feat_hbm, zero_hbm, out_hbm,
           acc_sh, sidx, didx, rows_a, rows_b, sem_a, sem_b):
    c = lax.axis_index("c")
    s = lax.axis_index("s")
    wid = c * _NS + s
    pltpu.sync_copy(zero_hbm.at[pl.ds(s * rpt, rpt)],
                    acc_sh.at[pl.ds(s * rpt, rpt)])
    pltpu.sync_copy(src_hbm.at[pl.ds(wid * k, k)], sidx)
    pltpu.sync_copy(dst_hbm.at[pl.ds(wid * k, k)], didx)
    plsc.subcore_barrier()

    pltpu.async_copy(feat_hbm.at[sidx.at[0]], rows_a, sem_a)
    k2 = k // 2

    def step2(jj, carry):
      j0 = 2 * jj
      j1 = j0 + 1
      pltpu.async_copy(feat_hbm.at[sidx.at[j1]], rows_b, sem_b)
      pltpu.make_async_copy(feat_hbm.at[sidx.at[j0]], rows_a, sem_a).wait()
      pltpu.sync_copy(rows_a, acc_sh.at[didx.at[j0]], add=True)

      @pl.when(jj + 1 < k2)
      def _():
        pltpu.async_copy(feat_hbm.at[sidx.at[j0 + 2]], rows_a, sem_a)

      pltpu.make_async_copy(feat_hbm.at[sidx.at[j1]], rows_b, sem_b).wait()
      pltpu.sync_copy(rows_b, acc_sh.at[didx.at[j1]], add=True)
      return carry

    lax.fori_loop(0, k2, step2, 0)
    plsc.subcore_barrier()
    pltpu.sync_copy(acc_sh.at[pl.ds(s * rpt, rpt)],
                    out_hbm.at[c, pl.ds(s * rpt, rpt)])

  run = pl.kernel(
      body,
      out_type=jax.ShapeDtypeStruct((_NC, n_pad, f), jnp.float32),
      mesh=mesh,
      scratch_types=[
          pltpu.VMEM_SHARED((n_pad, f), jnp.float32),
          pltpu.VMEM((k, ch), jnp.int32),
          pltpu.VMEM((k, ch), jnp.int32),
          pltpu.VMEM((ch, f), jnp.float32),
          pltpu.VMEM((ch, f), jnp.float32),
          pltpu.SemaphoreType.DMA,
          pltpu.SemaphoreType.DMA,
      ],
      compiler_params=pltpu.CompilerParams(use_tc_tiling_on_sc=False),
      interpret=interpret,
  )
  return run(src2d, dst2d, feat, zeros)


def _tc1(x, w1n, w1r, b1, bn, interpret=False):
  n, d = x.shape

  def body(x_ref, w1n_ref, w1r_ref, b1_ref, p1_ref, r1_ref):
    xb = x_ref[...]
    p = jnp.dot(xb, w1n_ref[...], preferred_element_type=jnp.float32)
    pad = jnp.concatenate(
        [jnp.ones((bn, 1), jnp.float32), jnp.zeros((bn, 15), jnp.float32)],
        axis=1)
    p1_ref[...] = jnp.concatenate([p, pad], axis=1)
    r1_ref[...] = (jnp.dot(xb, w1r_ref[...], preferred_element_type=jnp.float32)
                   + b1_ref[...])

  return pl.pallas_call(
      body,
      grid=(n // bn,),
      in_specs=[
          pl.BlockSpec((bn, d), lambda i: (i, 0)),
          pl.BlockSpec((d, d), lambda i: (0, 0)),
          pl.BlockSpec((d, d), lambda i: (0, 0)),
          pl.BlockSpec((1, d), lambda i: (0, 0)),
      ],
      out_specs=[
          pl.BlockSpec((bn, d + 16), lambda i: (i, 0)),
          pl.BlockSpec((bn, d), lambda i: (i, 0)),
      ],
      out_shape=[
          jax.ShapeDtypeStruct((n, d + 16), jnp.float32),
          jax.ShapeDtypeStruct((n, d), jnp.float32),
      ],
      interpret=interpret,
  )(x, w1n, w1r, b1.reshape(1, d))


def _tc2(agg1, r1, w2n, w2r, wcp, b2, bcp, bn, interpret=False):
  _, n_pad, f1 = agg1.shape
  n, d = r1.shape

  def body(agg_ref, r1_ref, w2n_ref, w2r_ref, wcp_ref, b2_ref, bcp_ref,
           q_ref, r2_ref, invd_ref):
    agg = agg_ref[0] + agg_ref[1]
    deg = agg[:, d:d + 1]
    invd = 1.0 / jnp.maximum(deg, 1.0)
    h1 = jnp.maximum(agg[:, :d] * invd + r1_ref[...], 0.0)
    w2nc = jnp.dot(w2n_ref[...], wcp_ref[...],
                   preferred_element_type=jnp.float32)
    w2rc = jnp.dot(w2r_ref[...], wcp_ref[...],
                   preferred_element_type=jnp.float32)
    bc2 = jnp.dot(b2_ref[...], wcp_ref[...],
                  preferred_element_type=jnp.float32) + bcp_ref[...]
    q_ref[...] = jnp.dot(h1, w2nc, preferred_element_type=jnp.float32)
    r2_ref[...] = jnp.dot(h1, w2rc, preferred_element_type=jnp.float32) + bc2
    invd_ref[...] = invd

  return pl.pallas_call(
      body,
      grid=(n // bn,),
      in_specs=[
          pl.BlockSpec((2, bn, f1), lambda i: (0, i, 0)),
          pl.BlockSpec((bn, d), lambda i: (i, 0)),
          pl.BlockSpec((d, d), lambda i: (0, 0)),
          pl.BlockSpec((d, d), lambda i: (0, 0)),
          pl.BlockSpec((d, 16), lambda i: (0, 0)),
          pl.BlockSpec((1, d), lambda i: (0, 0)),
          pl.BlockSpec((1, 16), lambda i: (0, 0)),
      ],
      out_specs=[
          pl.BlockSpec((bn, 16), lambda i: (i, 0)),
          pl.BlockSpec((bn, 16), lambda i: (i, 0)),
          pl.BlockSpec((bn, 1), lambda i: (i, 0)),
      ],
      out_shape=[
          jax.ShapeDtypeStruct((n, 16), jnp.float32),
          jax.ShapeDtypeStruct((n, 16), jnp.float32),
          jax.ShapeDtypeStruct((n, 1), jnp.float32),
      ],
      interpret=interpret,
  )(agg1, r1, w2n, w2r, wcp, b2.reshape(1, d), bcp.reshape(1, 16))


def _tc3(agg2, r2, invd, bn, interpret=False):
  _, n_pad, f2 = agg2.shape
  n = r2.shape[0]

  def body(agg_ref, r2_ref, invd_ref, out_ref):
    out_ref[...] = (agg_ref[0] + agg_ref[1]) * invd_ref[...] + r2_ref[...]

  return pl.pallas_call(
      body,
      grid=(n // bn,),
      in_specs=[
          pl.BlockSpec((2, bn, f2), lambda i: (0, i, 0)),
          pl.BlockSpec((bn, 16), lambda i: (i, 0)),
          pl.BlockSpec((bn, 1), lambda i: (i, 0)),
      ],
      out_specs=pl.BlockSpec((bn, 16), lambda i: (i, 0)),
      out_shape=jax.ShapeDtypeStruct((n, 16), jnp.float32),
      interpret=interpret,
  )(agg2, r2, invd)


def _impl(x, edge_index, w1_neigh, w1_root, b1, w2_neigh, w2_root, b2, wc, bc,
          interpret=False):
  n, d = x.shape
  e = edge_index.shape[1]
  c_out = wc.shape[1]

  ch_a, ch_c = _CH // 2, _CH
  chunk = ch_c * _NW * 2
  e_pad = -(-e // chunk) * chunk
  k_a = e_pad // (ch_a * _NW)
  k_c = e_pad // (ch_c * _NW)
  n_pad = -(-(n + 1) // (_NS * 8)) * (_NS * 8)
  f1 = d + 16

  src_flat = jnp.concatenate(
      [edge_index[0], jnp.zeros((e_pad - e,), jnp.int32)])
  dst_flat = jnp.concatenate(
      [edge_index[1], jnp.full((e_pad - e,), n, jnp.int32)])
  wcp = jnp.pad(wc, ((0, 0), (0, 16 - c_out)))
  bcp = jnp.pad(bc, (0, 16 - c_out))

  bn = 400 if n % 400 == 0 else 8 * (n // 8)

  p1aug, r1 = _tc1(x, w1_neigh, w1_root, b1, bn, interpret)
  agg1 = _sc_segment_sum(src_flat.reshape(_NW * k_a, ch_a),
                         dst_flat.reshape(_NW * k_a, ch_a),
                         p1aug, jnp.zeros((n_pad, f1), jnp.float32),
                         n_pad, f1, k_a, ch_a, interpret)
  q, r2, invd = _tc2(agg1, r1, w2_neigh, w2_root, wcp, b2, bcp, bn, interpret)
  agg2 = _sc_segment_sum(src_flat.reshape(_NW * k_c, ch_c),
                         dst_flat.reshape(_NW * k_c, ch_c),
                         q, jnp.zeros((n_pad, 16), jnp.float32),
                         n_pad, 16, k_c, ch_c, interpret)
  logits16 = _tc3(agg2, r2, invd, bn, interpret)
  return logits16[:, :c_out]


def kernel(x, edge_index, w1_neigh, w1_root, b1, w2_neigh, w2_root, b2, wc, bc):
  return _impl(x, edge_index, w1_neigh, w1_root, b1,
               w2_neigh, w2_root, b2, wc, bc)


# speedup vs baseline: 1.4946x; 1.4946x over previous
import functools

import jax
import jax.numpy as jnp
from jax import lax
from jax.experimental import pallas as pl
from jax.experimental.pallas import tpu as pltpu
from jax.experimental.pallas import tpu_sc as plsc

_NC = 2
_NS = 16
_NW = _NC * _NS
_CH = 128
_IB = 20


def _sc_mesh():
  return plsc.VectorSubcoreMesh(core_axis_name="c", subcore_axis_name="s",
                                num_cores=_NC, num_subcores=_NS)


def _sc_segment_sum_wide(src2d, dst2d, feat2, zeros, n_pad, f, k_t,
                         interpret=False):
  rpt = n_pad // _NS
  nb = k_t // _IB

  def body(src_hbm, dst_hbm, feat_hbm, zero_hbm, out_hbm,
           feat_sh, acc_sh, sidx, didx, rows, sem):
    c = lax.axis_index("c")
    s = lax.axis_index("s")
    pltpu.sync_copy(feat_hbm.at[c, pl.ds(s * rpt, rpt)],
                    feat_sh.at[pl.ds(s * rpt, rpt)])
    pltpu.sync_copy(zero_hbm.at[pl.ds(s * rpt, rpt)],
                    acc_sh.at[pl.ds(s * rpt, rpt)])
    plsc.subcore_barrier()

    def block(b, carry):
      base = s * k_t + b * _IB
      pltpu.sync_copy(src_hbm.at[pl.ds(base, _IB)], sidx)
      pltpu.sync_copy(dst_hbm.at[pl.ds(base, _IB)], didx)

      def step(jj, carry2):
        pltpu.async_copy(feat_sh.at[sidx.at[jj]], rows, sem).wait()
        pltpu.sync_copy(rows, acc_sh.at[didx.at[jj]], add=True)
        return carry2

      lax.fori_loop(0, _IB, step, 0)
      return carry

    lax.fori_loop(0, nb, block, 0)
    plsc.subcore_barrier()
    pltpu.sync_copy(acc_sh.at[pl.ds(s * rpt, rpt)],
                    out_hbm.at[c, pl.ds(s * rpt, rpt)])

  run = pl.kernel(
      body,
      out_type=jax.ShapeDtypeStruct((_NC, n_pad, f), jnp.float32),
      mesh=_sc_mesh(),
      scratch_types=[
          pltpu.VMEM_SHARED((n_pad, f), jnp.float32),
          pltpu.VMEM_SHARED((n_pad, f), jnp.float32),
          pltpu.VMEM((_IB, _CH), jnp.int32),
          pltpu.VMEM((_IB, _CH), jnp.int32),
          pltpu.VMEM((_CH, f), jnp.float32),
          pltpu.SemaphoreType.DMA,
      ],
      compiler_params=pltpu.CompilerParams(use_tc_tiling_on_sc=False),
      interpret=interpret,
  )
  return run(src2d, dst2d, feat2, zeros)


def _sc_segment_sum_narrow(src2d, dst2d, feat, zeros, n_pad, f, k,
                           interpret=False):
  rpt = n_pad // _NS

  def body(src_hbm, dst_hbm, feat_hbm, zero_hbm, out_hbm,
           feat_sh, acc_sh, sidx, didx, rows_a, rows_b, sem_a, sem_b):
    c = lax.axis_index("c")
    s = lax.axis_index("s")
    wid = c * _NS + s
    pltpu.sync_copy(feat_hbm.at[pl.ds(s * rpt, rpt)],
                    feat_sh.at[pl.ds(s * rpt, rpt)])
    pltpu.sync_copy(zero_hbm.at[pl.ds(s * rpt, rpt)],
                    acc_sh.at[pl.ds(s * rpt, rpt)])
    pltpu.sync_copy(src_hbm.at[pl.ds(wid * k, k)], sidx)
    pltpu.sync_copy(dst_hbm.at[pl.ds(wid * k, k)], didx)
    plsc.subcore_barrier()

    pltpu.async_copy(feat_sh.at[sidx.at[0]], rows_a, sem_a)
    k2 = k // 2

    def step2(jj, carry):
      j0 = 2 * jj
      j1 = j0 + 1
      pltpu.async_copy(feat_sh.at[sidx.at[j1]], rows_b, sem_b)
      pltpu.make_async_copy(feat_sh.at[sidx.at[j0]], rows_a, sem_a).wait()
      pltpu.sync_copy(rows_a, acc_sh.at[didx.at[j0]], add=True)

      @pl.when(jj + 1 < k2)
      def _():
        pltpu.async_copy(feat_sh.at[sidx.at[j0 + 2]], rows_a, sem_a)

      pltpu.make_async_copy(feat_sh.at[sidx.at[j1]], rows_b, sem_b).wait()
      pltpu.sync_copy(rows_b, acc_sh.at[didx.at[j1]], add=True)
      return carry

    lax.fori_loop(0, k2, step2, 0)
    plsc.subcore_barrier()
    pltpu.sync_copy(acc_sh.at[pl.ds(s * rpt, rpt)],
                    out_hbm.at[c, pl.ds(s * rpt, rpt)])

  run = pl.kernel(
      body,
      out_type=jax.ShapeDtypeStruct((_NC, n_pad, f), jnp.float32),
      mesh=_sc_mesh(),
      scratch_types=[
          pltpu.VMEM_SHARED((n_pad, f), jnp.float32),
          pltpu.VMEM_SHARED((n_pad, f), jnp.float32),
          pltpu.VMEM((k, _CH), jnp.int32),
          pltpu.VMEM((k, _CH), jnp.int32),
          pltpu.VMEM((_CH, f), jnp.float32),
          pltpu.VMEM((_CH, f), jnp.float32),
          pltpu.SemaphoreType.DMA,
          pltpu.SemaphoreType.DMA,
      ],
      compiler_params=pltpu.CompilerParams(use_tc_tiling_on_sc=False),
      interpret=interpret,
  )
  return run(src2d, dst2d, feat, zeros)


def _tc1(x, w1n, w1r, b1, bn, interpret=False):
  n, d = x.shape

  def body(x_ref, w1n_ref, w1r_ref, b1_ref, p1_ref, r1_ref):
    xb = x_ref[...]
    p1_ref[...] = jnp.dot(xb, w1n_ref[...], preferred_element_type=jnp.float32)
    r1_ref[...] = (jnp.dot(xb, w1r_ref[...], preferred_element_type=jnp.float32)
                   + b1_ref[...])

  return pl.pallas_call(
      body,
      grid=(n // bn,),
      in_specs=[
          pl.BlockSpec((bn, d), lambda i: (i, 0)),
          pl.BlockSpec((d, d), lambda i: (0, 0)),
          pl.BlockSpec((d, d), lambda i: (0, 0)),
          pl.BlockSpec((1, d), lambda i: (0, 0)),
      ],
      out_specs=[
          pl.BlockSpec((bn, d), lambda i: (i, 0)),
          pl.BlockSpec((bn, d), lambda i: (i, 0)),
      ],
      out_shape=[
          jax.ShapeDtypeStruct((n, d), jnp.float32),
          jax.ShapeDtypeStruct((n, d), jnp.float32),
      ],
      interpret=interpret,
  )(x, w1n, w1r, b1.reshape(1, d))


def _tc2(agg1, r1, w2n, w2r, wcp, b2, bcp, bn, interpret=False):
  _, n_pad, fh = agg1.shape
  n, d = r1.shape
  dh = d // 2

  def body(agg_ref, r1_ref, w2n_ref, w2r_ref, wcp_ref, b2_ref, bcp_ref,
           q_ref, r2_ref, invd_ref):
    lo = agg_ref[0]
    hi = agg_ref[1]
    deg = lo[:, dh:dh + 1]
    invd = 1.0 / jnp.maximum(deg, 1.0)
    agg = jnp.concatenate([lo[:, :dh], hi[:, :dh]], axis=1)
    h1 = jnp.maximum(agg * invd + r1_ref[...], 0.0)
    w2nc = jnp.dot(w2n_ref[...], wcp_ref[...],
                   preferred_element_type=jnp.float32)
    w2rc = jnp.dot(w2r_ref[...], wcp_ref[...],
                   preferred_element_type=jnp.float32)
    bc2 = jnp.dot(b2_ref[...], wcp_ref[...],
                  preferred_element_type=jnp.float32) + bcp_ref[...]
    q_ref[...] = jnp.dot(h1, w2nc, preferred_element_type=jnp.float32)
    r2_ref[...] = jnp.dot(h1, w2rc, preferred_element_type=jnp.float32) + bc2
    invd_ref[...] = invd

  return pl.pallas_call(
      body,
      grid=(n // bn,),
      in_specs=[
          pl.BlockSpec((2, bn, fh), lambda i: (0, i, 0)),
          pl.BlockSpec((bn, d), lambda i: (i, 0)),
          pl.BlockSpec((d, d), lambda i: (0, 0)),
          pl.BlockSpec((d, d), lambda i: (0, 0)),
          pl.BlockSpec((d, 16), lambda i: (0, 0)),
          pl.BlockSpec((1, d), lambda i: (0, 0)),
          pl.BlockSpec((1, 16), lambda i: (0, 0)),
      ],
      out_specs=[
          pl.BlockSpec((bn, 16), lambda i: (i, 0)),
          pl.BlockSpec((bn, 16), lambda i: (i, 0)),
          pl.BlockSpec((bn, 1), lambda i: (i, 0)),
      ],
      out_shape=[
          jax.ShapeDtypeStruct((n, 16), jnp.float32),
          jax.ShapeDtypeStruct((n, 16), jnp.float32),
          jax.ShapeDtypeStruct((n, 1), jnp.float32),
      ],
      interpret=interpret,
  )(agg1, r1, w2n, w2r, wcp, b2.reshape(1, d), bcp.reshape(1, 16))


def _tc3(agg2, r2, invd, bn, interpret=False):
  _, n_pad, f2 = agg2.shape
  n = r2.shape[0]

  def body(agg_ref, r2_ref, invd_ref, out_ref):
    out_ref[...] = (agg_ref[0] + agg_ref[1]) * invd_ref[...] + r2_ref[...]

  return pl.pallas_call(
      body,
      grid=(n // bn,),
      in_specs=[
          pl.BlockSpec((2, bn, f2), lambda i: (0, i, 0)),
          pl.BlockSpec((bn, 16), lambda i: (i, 0)),
          pl.BlockSpec((bn, 1), lambda i: (i, 0)),
      ],
      out_specs=pl.BlockSpec((bn, 16), lambda i: (i, 0)),
      out_shape=jax.ShapeDtypeStruct((n, 16), jnp.float32),
      interpret=interpret,
  )(agg2, r2, invd)


def _impl(x, edge_index, w1_neigh, w1_root, b1, w2_neigh, w2_root, b2, wc, bc,
          interpret=False):
  n, d = x.shape
  e = edge_index.shape[1]
  c_out = wc.shape[1]
  dh = d // 2
  fh = dh + 16

  chunk = _NS * _IB * _CH
  e_pad = -(-e // chunk) * chunk
  k_t = e_pad // (_NS * _CH)
  k = e_pad // (_NW * _CH)
  n_pad = -(-(n + 1) // (_NS * 8)) * (_NS * 8)

  src_flat = jnp.concatenate(
      [edge_index[0], jnp.zeros((e_pad - e,), jnp.int32)])
  dst_flat = jnp.concatenate(
      [edge_index[1], jnp.full((e_pad - e,), n, jnp.int32)])
  src2d = src_flat.reshape(_NS * k_t, _CH)
  dst2d = dst_flat.reshape(_NS * k_t, _CH)
  wcp = jnp.pad(wc, ((0, 0), (0, 16 - c_out)))
  bcp = jnp.pad(bc, (0, 16 - c_out))

  bn = 400 if n % 400 == 0 else 8 * (n // 8)

  p1, r1 = _tc1(x, w1_neigh, w1_root, b1, bn, interpret)
  ones = jnp.ones((n, 1), jnp.float32)
  zer = jnp.zeros((n, fh - dh - 1), jnp.float32)
  feat2 = jnp.stack([
      jnp.concatenate([p1[:, :dh], ones, zer], axis=1),
      jnp.concatenate([p1[:, dh:], ones, zer], axis=1),
  ])
  feat2 = jnp.pad(feat2, ((0, 0), (0, n_pad - n), (0, 0)))

  agg1 = _sc_segment_sum_wide(src2d, dst2d, feat2,
                              jnp.zeros((n_pad, fh), jnp.float32),
                              n_pad, fh, k_t, interpret)
  q, r2, invd = _tc2(agg1, r1, w2_neigh, w2_root, wcp, b2, bcp, bn, interpret)
  qp = jnp.pad(q, ((0, n_pad - n), (0, 0)))
  agg2 = _sc_segment_sum_narrow(src2d, dst2d, qp,
                                jnp.zeros((n_pad, 16), jnp.float32),
                                n_pad, 16, k, interpret)
  logits16 = _tc3(agg2, r2, invd, bn, interpret)
  return logits16[:, :c_out]


def kernel(x, edge_index, w1_neigh, w1_root, b1, w2_neigh, w2_root, b2, wc, bc):
  return _impl(x, edge_index, w1_neigh, w1_root, b1,
               w2_neigh, w2_root, b2, wc, bc)


# speedup vs baseline: 1.7867x; 1.1954x over previous
import functools

import jax
import jax.numpy as jnp
from jax import lax
from jax.experimental import pallas as pl
from jax.experimental.pallas import tpu as pltpu
from jax.experimental.pallas import tpu_sc as plsc

_NC = 2
_NS = 16
_NW = _NC * _NS
_CH = 128
_IB = 20


def _sc_mesh():
  return plsc.VectorSubcoreMesh(core_axis_name="c", subcore_axis_name="s",
                                num_cores=_NC, num_subcores=_NS)


def _sc_segment_sum_wide(src2d, dst2d, feat_a, feat_b, zeros, n_pad, f, k_t,
                         interpret=False):
  rpt = n_pad // _NS
  nb = k_t // _IB
  p2 = _IB // 2

  def body(src_hbm, dst_hbm, feat_a_hbm, feat_b_hbm, zero_hbm, out_hbm,
           feat_sh, acc_sh, sidx, didx, rows_a, rows_b, sem_a, sem_b):
    c = lax.axis_index("c")
    s = lax.axis_index("s")
    sl = pl.ds(s * rpt, rpt)

    @pl.when(c == 0)
    def _():
      pltpu.sync_copy(feat_a_hbm.at[sl], feat_sh.at[sl])

    @pl.when(c == 1)
    def _():
      pltpu.sync_copy(feat_b_hbm.at[sl], feat_sh.at[sl])

    pltpu.sync_copy(zero_hbm.at[sl], acc_sh.at[sl])
    plsc.subcore_barrier()

    def block(b, carry):
      base = s * k_t + b * _IB
      pltpu.sync_copy(src_hbm.at[pl.ds(base, _IB)], sidx)
      pltpu.sync_copy(dst_hbm.at[pl.ds(base, _IB)], didx)
      pltpu.async_copy(feat_sh.at[sidx.at[0]], rows_a, sem_a)

      def pair(jj, carry2):
        j0 = 2 * jj
        j1 = j0 + 1
        pltpu.async_copy(feat_sh.at[sidx.at[j1]], rows_b, sem_b)
        pltpu.make_async_copy(feat_sh.at[sidx.at[j0]], rows_a, sem_a).wait()
        pltpu.sync_copy(rows_a, acc_sh.at[didx.at[j0]], add=True)

        @pl.when(jj + 1 < p2)
        def _():
          pltpu.async_copy(feat_sh.at[sidx.at[j0 + 2]], rows_a, sem_a)

        pltpu.make_async_copy(feat_sh.at[sidx.at[j1]], rows_b, sem_b).wait()
        pltpu.sync_copy(rows_b, acc_sh.at[didx.at[j1]], add=True)
        return carry2

      lax.fori_loop(0, p2, pair, 0)
      return carry

    lax.fori_loop(0, nb, block, 0)
    plsc.subcore_barrier()
    pltpu.sync_copy(acc_sh.at[sl], out_hbm.at[c, sl])

  run = pl.kernel(
      body,
      out_type=jax.ShapeDtypeStruct((_NC, n_pad, f), jnp.float32),
      mesh=_sc_mesh(),
      scratch_types=[
          pltpu.VMEM_SHARED((n_pad, f), jnp.float32),
          pltpu.VMEM_SHARED((n_pad, f), jnp.float32),
          pltpu.VMEM((_IB, _CH), jnp.int32),
          pltpu.VMEM((_IB, _CH), jnp.int32),
          pltpu.VMEM((_CH, f), jnp.float32),
          pltpu.VMEM((_CH, f), jnp.float32),
          pltpu.SemaphoreType.DMA,
          pltpu.SemaphoreType.DMA,
      ],
      compiler_params=pltpu.CompilerParams(use_tc_tiling_on_sc=False),
      interpret=interpret,
  )
  return run(src2d, dst2d, feat_a, feat_b, zeros)


def _sc_segment_sum_narrow(src2d, dst2d, feat, zeros, n_pad, f, k,
                           interpret=False):
  rpt = n_pad // _NS

  def body(src_hbm, dst_hbm, feat_hbm, zero_hbm, out_hbm,
           feat_sh, acc_sh, sidx, didx, rows_a, rows_b, sem_a, sem_b):
    c = lax.axis_index("c")
    s = lax.axis_index("s")
    wid = c * _NS + s
    sl = pl.ds(s * rpt, rpt)
    pltpu.sync_copy(feat_hbm.at[sl], feat_sh.at[sl])
    pltpu.sync_copy(zero_hbm.at[sl], acc_sh.at[sl])
    pltpu.sync_copy(src_hbm.at[pl.ds(wid * k, k)], sidx)
    pltpu.sync_copy(dst_hbm.at[pl.ds(wid * k, k)], didx)
    plsc.subcore_barrier()

    pltpu.async_copy(feat_sh.at[sidx.at[0]], rows_a, sem_a)
    k2 = k // 2

    def step2(jj, carry):
      j0 = 2 * jj
      j1 = j0 + 1
      pltpu.async_copy(feat_sh.at[sidx.at[j1]], rows_b, sem_b)
      pltpu.make_async_copy(feat_sh.at[sidx.at[j0]], rows_a, sem_a).wait()
      pltpu.sync_copy(rows_a, acc_sh.at[didx.at[j0]], add=True)

      @pl.when(jj + 1 < k2)
      def _():
        pltpu.async_copy(feat_sh.at[sidx.at[j0 + 2]], rows_a, sem_a)

      pltpu.make_async_copy(feat_sh.at[sidx.at[j1]], rows_b, sem_b).wait()
      pltpu.sync_copy(rows_b, acc_sh.at[didx.at[j1]], add=True)
      return carry

    lax.fori_loop(0, k2, step2, 0)
    plsc.subcore_barrier()
    pltpu.sync_copy(acc_sh.at[sl], out_hbm.at[c, sl])

  run = pl.kernel(
      body,
      out_type=jax.ShapeDtypeStruct((_NC, n_pad, f), jnp.float32),
      mesh=_sc_mesh(),
      scratch_types=[
          pltpu.VMEM_SHARED((n_pad, f), jnp.float32),
          pltpu.VMEM_SHARED((n_pad, f), jnp.float32),
          pltpu.VMEM((k, _CH), jnp.int32),
          pltpu.VMEM((k, _CH), jnp.int32),
          pltpu.VMEM((_CH, f), jnp.float32),
          pltpu.VMEM((_CH, f), jnp.float32),
          pltpu.SemaphoreType.DMA,
          pltpu.SemaphoreType.DMA,
      ],
      compiler_params=pltpu.CompilerParams(use_tc_tiling_on_sc=False),
      interpret=interpret,
  )
  return run(src2d, dst2d, feat, zeros)


def _tc1(x, w1n, w1r, b1, w2n, w2r, wcp, b2, bcp, n_pad, fh, bn,
         interpret=False):
  n, d = x.shape
  dh = d // 2

  def body(x_ref, w1n_ref, w1r_ref, b1_ref, w2n_ref, w2r_ref, wcp_ref,
           b2_ref, bcp_ref, fa_ref, fb_ref, r1_ref, w2nc_ref, w2rc_ref,
           bc2_ref):
    xb = x_ref[...]
    p = jnp.dot(xb, w1n_ref[...], preferred_element_type=jnp.float32)
    ones = jnp.ones((bn, 1), jnp.float32)
    zer = jnp.zeros((bn, fh - dh - 1), jnp.float32)
    fa_ref[...] = jnp.concatenate([p[:, :dh], ones, zer], axis=1)
    fb_ref[...] = jnp.concatenate([p[:, dh:], ones, zer], axis=1)
    r1_ref[...] = (jnp.dot(xb, w1r_ref[...], preferred_element_type=jnp.float32)
                   + b1_ref[...])
    w2nc_ref[...] = jnp.dot(w2n_ref[...], wcp_ref[...],
                            preferred_element_type=jnp.float32)
    w2rc_ref[...] = jnp.dot(w2r_ref[...], wcp_ref[...],
                            preferred_element_type=jnp.float32)
    bc2_ref[...] = jnp.dot(b2_ref[...], wcp_ref[...],
                           preferred_element_type=jnp.float32) + bcp_ref[...]

  return pl.pallas_call(
      body,
      grid=(n // bn,),
      in_specs=[
          pl.BlockSpec((bn, d), lambda i: (i, 0)),
          pl.BlockSpec((d, d), lambda i: (0, 0)),
          pl.BlockSpec((d, d), lambda i: (0, 0)),
          pl.BlockSpec((1, d), lambda i: (0, 0)),
          pl.BlockSpec((d, d), lambda i: (0, 0)),
          pl.BlockSpec((d, d), lambda i: (0, 0)),
          pl.BlockSpec((d, 16), lambda i: (0, 0)),
          pl.BlockSpec((1, d), lambda i: (0, 0)),
          pl.BlockSpec((1, 16), lambda i: (0, 0)),
      ],
      out_specs=[
          pl.BlockSpec((bn, fh), lambda i: (i, 0)),
          pl.BlockSpec((bn, fh), lambda i: (i, 0)),
          pl.BlockSpec((bn, d), lambda i: (i, 0)),
          pl.BlockSpec((d, 16), lambda i: (0, 0)),
          pl.BlockSpec((d, 16), lambda i: (0, 0)),
          pl.BlockSpec((1, 16), lambda i: (0, 0)),
      ],
      out_shape=[
          jax.ShapeDtypeStruct((n_pad, fh), jnp.float32),
          jax.ShapeDtypeStruct((n_pad, fh), jnp.float32),
          jax.ShapeDtypeStruct((n, d), jnp.float32),
          jax.ShapeDtypeStruct((d, 16), jnp.float32),
          jax.ShapeDtypeStruct((d, 16), jnp.float32),
          jax.ShapeDtypeStruct((1, 16), jnp.float32),
      ],
      interpret=interpret,
  )(x, w1n, w1r, b1.reshape(1, d), w2n, w2r, wcp, b2.reshape(1, d),
    bcp.reshape(1, 16))


def _tc2(agg1, r1, w2nc, w2rc, bc2, bn, interpret=False):
  _, n_pad, fh = agg1.shape
  n, d = r1.shape
  dh = d // 2

  def body(agg_ref, r1_ref, w2nc_ref, w2rc_ref, bc2_ref,
           q_ref, r2_ref, invd_ref):
    lo = agg_ref[0]
    hi = agg_ref[1]
    deg = lo[:, dh:dh + 1]
    invd = 1.0 / jnp.maximum(deg, 1.0)
    agg = jnp.concatenate([lo[:, :dh], hi[:, :dh]], axis=1)
    h1 = jnp.maximum(agg * invd + r1_ref[...], 0.0)
    q_ref[...] = jnp.dot(h1, w2nc_ref[...], preferred_element_type=jnp.float32)
    r2_ref[...] = (jnp.dot(h1, w2rc_ref[...],
                           preferred_element_type=jnp.float32) + bc2_ref[...])
    invd_ref[...] = invd

  return pl.pallas_call(
      body,
      grid=(n // bn,),
      in_specs=[
          pl.BlockSpec((2, bn, fh), lambda i: (0, i, 0)),
          pl.BlockSpec((bn, d), lambda i: (i, 0)),
          pl.BlockSpec((d, 16), lambda i: (0, 0)),
          pl.BlockSpec((d, 16), lambda i: (0, 0)),
          pl.BlockSpec((1, 16), lambda i: (0, 0)),
      ],
      out_specs=[
          pl.BlockSpec((bn, 16), lambda i: (i, 0)),
          pl.BlockSpec((bn, 16), lambda i: (i, 0)),
          pl.BlockSpec((bn, 1), lambda i: (i, 0)),
      ],
      out_shape=[
          jax.ShapeDtypeStruct((n, 16), jnp.float32),
          jax.ShapeDtypeStruct((n, 16), jnp.float32),
          jax.ShapeDtypeStruct((n, 1), jnp.float32),
      ],
      interpret=interpret,
  )(agg1, r1, w2nc, w2rc, bc2)


def _tc3(agg2, r2, invd, c_out, bn, interpret=False):
  _, n_pad, f2 = agg2.shape
  n = r2.shape[0]

  def body(agg_ref, r2_ref, invd_ref, out_ref):
    v = (agg_ref[0] + agg_ref[1]) * invd_ref[...] + r2_ref[...]
    out_ref[...] = v[:, :c_out]

  return pl.pallas_call(
      body,
      grid=(n // bn,),
      in_specs=[
          pl.BlockSpec((2, bn, f2), lambda i: (0, i, 0)),
          pl.BlockSpec((bn, 16), lambda i: (i, 0)),
          pl.BlockSpec((bn, 1), lambda i: (i, 0)),
      ],
      out_specs=pl.BlockSpec((bn, c_out), lambda i: (i, 0)),
      out_shape=jax.ShapeDtypeStruct((n, c_out), jnp.float32),
      interpret=interpret,
  )(agg2, r2, invd)


def _impl(x, edge_index, w1_neigh, w1_root, b1, w2_neigh, w2_root, b2, wc, bc,
          interpret=False):
  n, d = x.shape
  e = edge_index.shape[1]
  c_out = wc.shape[1]
  dh = d // 2
  fh = dh + 16

  chunk = _NS * _IB * _CH
  e_pad = -(-e // chunk) * chunk
  k_t = e_pad // (_NS * _CH)
  k = e_pad // (_NW * _CH)
  n_pad = -(-(n + 1) // (_NS * 8)) * (_NS * 8)

  src_flat = jnp.concatenate(
      [edge_index[0], jnp.zeros((e_pad - e,), jnp.int32)])
  dst_flat = jnp.concatenate(
      [edge_index[1], jnp.full((e_pad - e,), n, jnp.int32)])
  src2d = src_flat.reshape(_NS * k_t, _CH)
  dst2d = dst_flat.reshape(_NS * k_t, _CH)
  wcp = jnp.pad(wc, ((0, 0), (0, 16 - c_out)))
  bcp = jnp.pad(bc, (0, 16 - c_out))

  bn = 400 if n % 400 == 0 else 8 * (n // 8)

  feat_a, feat_b, r1, w2nc, w2rc, bc2 = _tc1(
      x, w1_neigh, w1_root, b1, w2_neigh, w2_root, wcp, b2, bcp,
      n_pad, fh, bn, interpret)
  agg1 = _sc_segment_sum_wide(src2d, dst2d, feat_a, feat_b,
                              jnp.zeros((n_pad, fh), jnp.float32),
                              n_pad, fh, k_t, interpret)
  q, r2, invd = _tc2(agg1, r1, w2nc, w2rc, bc2, bn, interpret)
  qp = jnp.pad(q, ((0, n_pad - n), (0, 0)))
  agg2 = _sc_segment_sum_narrow(src2d, dst2d, qp,
                                jnp.zeros((n_pad, 16), jnp.float32),
                                n_pad, 16, k, interpret)
  return _tc3(agg2, r2, invd, c_out, bn, interpret)


def kernel(x, edge_index, w1_neigh, w1_root, b1, w2_neigh, w2_root, b2, wc, bc):
  return _impl(x, edge_index, w1_neigh, w1_root, b1,
               w2_neigh, w2_root, b2, wc, bc)
